# baseline (speedup 1.0000x reference)
import functools
import jax
import jax.numpy as jnp
from jax import lax
from jax.experimental import pallas as pl
from jax.experimental.pallas import tpu as pltpu
from jax.experimental.pallas import tpu_sc as plsc

B, N, K, D = 1024, 500, 50, 128
R = 2 + K
RPAD = 64
L = 16
HB = B // 2
GB = 128
NH = 250


def _sc_gather(half, nodes_hbm, first_hbm, last_hbm, cand_hbm,
               gxa_hbm, gya_hbm, gxb_hbm, gyb_hbm,
               nodes_v, idx_v, g_v):
    wid = lax.axis_index("s") * 2 + lax.axis_index("c")

    @pl.when(wid < 16)
    def _():
        p = wid % 2
        g = (wid // 2) % 4
        nh = wid // 8
        bl = g * GB
        b0 = half * HB + bl
        i0 = nh * NH
        pltpu.sync_copy(nodes_hbm.at[pl.ds(i0, NH), p, pl.ds(b0, GB)],
                        nodes_v)
        pltpu.sync_copy(first_hbm.at[pl.ds(b0, GB)], idx_v.at[0])
        pltpu.sync_copy(last_hbm.at[pl.ds(b0, GB)], idx_v.at[1])
        pltpu.sync_copy(cand_hbm.at[:, pl.ds(b0, GB)], idx_v.at[pl.ds(2, K)])
        bbase = lax.iota(jnp.int32, L)

        @plsc.parallel_loop(0, R)
        def _gather(j):
            for bb in range(GB // L):
                ids = idx_v[j, pl.ds(bb * L, L)] - i0
                valid = (ids >= 0) & (ids < NH)
                safe = jnp.clip(ids, 0, NH - 1)
                vals = plsc.load_gather(nodes_v, [safe, bbase + bb * L])
                g_v[j, pl.ds(bb * L, L)] = jnp.where(valid, vals, 0.0)

        @pl.when((p == 0) & (nh == 0))
        def _():
            pltpu.sync_copy(g_v, gxa_hbm.at[:, pl.ds(bl, GB)])

        @pl.when((p == 1) & (nh == 0))
        def _():
            pltpu.sync_copy(g_v, gya_hbm.at[:, pl.ds(bl, GB)])

        @pl.when((p == 0) & (nh == 1))
        def _():
            pltpu.sync_copy(g_v, gxb_hbm.at[:, pl.ds(bl, GB)])

        @pl.when((p == 1) & (nh == 1))
        def _():
            pltpu.sync_copy(g_v, gyb_hbm.at[:, pl.ds(bl, GB)])


def _make_sc_call(half):
    return pl.kernel(
        functools.partial(_sc_gather, half),
        out_type=tuple(
            jax.ShapeDtypeStruct((RPAD, HB), jnp.float32) for _ in range(4)),
        mesh=plsc.VectorSubcoreMesh(core_axis_name="c", subcore_axis_name="s"),
        compiler_params=pltpu.CompilerParams(needs_layout_passes=False),
        scratch_types=[
            pltpu.VMEM((NH, GB), jnp.float32),
            pltpu.VMEM((RPAD, GB), jnp.int32),
            pltpu.VMEM((RPAD, GB), jnp.float32),
        ],
    )


_sc_calls = (_make_sc_call(0), _make_sc_call(1))


TB = 256
NSTEP = HB // TB


def _tc_body(gxa_ref, gya_ref, gxb_ref, gyb_ref,
             wrows_ref, w1_ref, w2_ref, out_ref):
    gxt = gxa_ref[...] + gxb_ref[...]
    gyt = gya_ref[...] + gyb_ref[...]
    wx = wrows_ref[0, :]
    wy = wrows_ref[1, :]
    bc = wrows_ref[2, :]
    w1b = wrows_ref[3, :]
    w2b = wrows_ref[4, :]
    emb = gxt[:, :, None] * wx[None, None, :] \
        + gyt[:, :, None] * wy[None, None, :] + bc[None, None, :]
    f = jnp.dot(emb[0], w1_ref[...],
                preferred_element_type=jnp.float32) + w1b[None, :]
    l = jnp.dot(emb[1], w2_ref[...],
                preferred_element_type=jnp.float32) + w2b[None, :]
    out_ref[...] = jnp.concatenate(
        [f[None, :, :], l[None, :, :], emb[2:R]], axis=0)


def _tc_dense0(gxa_ref, gya_ref, gxb_ref, gyb_ref,
               wrows_ref, w1_ref, w2_ref, out_ref):
    _tc_body(gxa_ref, gya_ref, gxb_ref, gyb_ref,
             wrows_ref, w1_ref, w2_ref, out_ref)


def _tc_dense1(buf_ref, gxa_ref, gya_ref, gxb_ref, gyb_ref,
               wrows_ref, w1_ref, w2_ref, out_ref):
    _tc_body(gxa_ref, gya_ref, gxb_ref, gyb_ref,
             wrows_ref, w1_ref, w2_ref, out_ref)


_common_in_specs = [
    pl.BlockSpec((RPAD, TB), lambda i: (0, i)),
    pl.BlockSpec((RPAD, TB), lambda i: (0, i)),
    pl.BlockSpec((RPAD, TB), lambda i: (0, i)),
    pl.BlockSpec((RPAD, TB), lambda i: (0, i)),
    pl.BlockSpec((8, D), lambda i: (0, 0)),
    pl.BlockSpec((D, D), lambda i: (0, 0)),
    pl.BlockSpec((D, D), lambda i: (0, 0)),
]


def kernel(nodes, first_node_idx, last_node_idx, candidate_indices,
           W_coord, b_coord, W1_w, W1_b, W2_w, W2_b):
    nodes_t = jnp.transpose(nodes, (1, 2, 0))
    cand_t = jnp.transpose(candidate_indices, (1, 0))
    wrows = jnp.concatenate(
        [W_coord, b_coord[None], W1_b[None], W2_b[None],
         jnp.zeros((3, D), jnp.float32)], axis=0)

    g0 = _sc_calls[0](nodes_t, first_node_idx, last_node_idx, cand_t)
    out_half0 = pl.pallas_call(
        _tc_dense0,
        grid=(NSTEP,),
        in_specs=_common_in_specs,
        out_specs=pl.BlockSpec((R, TB, D), lambda i: (0, i, 0)),
        out_shape=jax.ShapeDtypeStruct((R, B, D), jnp.float32),
    )(*g0, wrows, W1_w, W2_w)

    g1 = _sc_calls[1](nodes_t, first_node_idx, last_node_idx, cand_t)
    out_t = pl.pallas_call(
        _tc_dense1,
        grid=(NSTEP,),
        in_specs=[pl.BlockSpec(memory_space=pl.ANY)] + _common_in_specs,
        out_specs=pl.BlockSpec((R, TB, D), lambda i: (0, i + NSTEP, 0)),
        out_shape=jax.ShapeDtypeStruct((R, B, D), jnp.float32),
        input_output_aliases={0: 0},
    )(out_half0, *g1, wrows, W1_w, W2_w)

    return jnp.transpose(out_t, (1, 0, 2))


# speedup vs baseline: 1.0453x; 1.0453x over previous
import functools
import jax
import jax.numpy as jnp
from jax import lax
from jax.experimental import pallas as pl
from jax.experimental.pallas import tpu as pltpu
from jax.experimental.pallas import tpu_sc as plsc

B, N, K, D = 1024, 500, 50, 128
R = 2 + K
RPAD = 64
L = 16
HB = B // 2
GB = 128
NH = 250


def _sc_gather(half, nodes_hbm, first_hbm, last_hbm, cand_hbm,
               gxa_hbm, gya_hbm, gxb_hbm, gyb_hbm,
               nodes_v, idx_v, g_v):
    wid = lax.axis_index("s") * 2 + lax.axis_index("c")

    @pl.when(wid < 16)
    def _():
        p = wid % 2
        g = (wid // 2) % 4
        nh = wid // 8
        bl = g * GB
        b0 = half * HB + bl
        i0 = nh * NH
        pltpu.sync_copy(nodes_hbm.at[pl.ds(i0, NH), p, pl.ds(b0, GB)],
                        nodes_v)
        pltpu.sync_copy(first_hbm.at[pl.ds(b0, GB)], idx_v.at[0])
        pltpu.sync_copy(last_hbm.at[pl.ds(b0, GB)], idx_v.at[1])
        pltpu.sync_copy(cand_hbm.at[:, pl.ds(b0, GB)], idx_v.at[pl.ds(2, K)])
        bbase = lax.iota(jnp.int32, L)

        @plsc.parallel_loop(0, R)
        def _gather(j):
            for bb in range(GB // L):
                ids = idx_v[j, pl.ds(bb * L, L)] - i0
                valid = (ids >= 0) & (ids < NH)
                safe = jnp.clip(ids, 0, NH - 1)
                vals = plsc.load_gather(nodes_v, [safe, bbase + bb * L])
                g_v[j, pl.ds(bb * L, L)] = jnp.where(valid, vals, 0.0)

        @pl.when((p == 0) & (nh == 0))
        def _():
            pltpu.sync_copy(g_v, gxa_hbm.at[:, pl.ds(bl, GB)])

        @pl.when((p == 1) & (nh == 0))
        def _():
            pltpu.sync_copy(g_v, gya_hbm.at[:, pl.ds(bl, GB)])

        @pl.when((p == 0) & (nh == 1))
        def _():
            pltpu.sync_copy(g_v, gxb_hbm.at[:, pl.ds(bl, GB)])

        @pl.when((p == 1) & (nh == 1))
        def _():
            pltpu.sync_copy(g_v, gyb_hbm.at[:, pl.ds(bl, GB)])


def _make_sc_call(half):
    return pl.kernel(
        functools.partial(_sc_gather, half),
        out_type=tuple(
            jax.ShapeDtypeStruct((RPAD, HB), jnp.float32) for _ in range(4)),
        mesh=plsc.VectorSubcoreMesh(core_axis_name="c", subcore_axis_name="s"),
        compiler_params=pltpu.CompilerParams(needs_layout_passes=False),
        scratch_types=[
            pltpu.VMEM((NH, GB), jnp.float32),
            pltpu.VMEM((RPAD, GB), jnp.int32),
            pltpu.VMEM((RPAD, GB), jnp.float32),
        ],
    )


_sc_calls = (_make_sc_call(0), _make_sc_call(1))


TB = 128
NSTEP = HB // TB


def _tc_body(gxa_ref, gya_ref, gxb_ref, gyb_ref,
             wrows_ref, w1_ref, w2_ref, out_ref):
    gxt = gxa_ref[:R, :] + gxb_ref[:R, :]
    gyt = gya_ref[:R, :] + gyb_ref[:R, :]
    wx = wrows_ref[0, :]
    wy = wrows_ref[1, :]
    bc = wrows_ref[2, :]
    w1b = wrows_ref[3, :]
    w2b = wrows_ref[4, :]
    emb = gxt[:, :, None] * wx[None, None, :] \
        + gyt[:, :, None] * wy[None, None, :] + bc[None, None, :]
    f = jnp.dot(emb[0], w1_ref[...],
                preferred_element_type=jnp.float32) + w1b[None, :]
    l = jnp.dot(emb[1], w2_ref[...],
                preferred_element_type=jnp.float32) + w2b[None, :]
    out_ref[...] = jnp.concatenate(
        [f[None, :, :], l[None, :, :], emb[2:R]], axis=0)


def _tc_dense0(gxa_ref, gya_ref, gxb_ref, gyb_ref,
               wrows_ref, w1_ref, w2_ref, out_ref):
    _tc_body(gxa_ref, gya_ref, gxb_ref, gyb_ref,
             wrows_ref, w1_ref, w2_ref, out_ref)


def _tc_dense1(buf_ref, gxa_ref, gya_ref, gxb_ref, gyb_ref,
               wrows_ref, w1_ref, w2_ref, out_ref):
    _tc_body(gxa_ref, gya_ref, gxb_ref, gyb_ref,
             wrows_ref, w1_ref, w2_ref, out_ref)


_common_in_specs = [
    pl.BlockSpec((RPAD, TB), lambda i: (0, i)),
    pl.BlockSpec((RPAD, TB), lambda i: (0, i)),
    pl.BlockSpec((RPAD, TB), lambda i: (0, i)),
    pl.BlockSpec((RPAD, TB), lambda i: (0, i)),
    pl.BlockSpec((8, D), lambda i: (0, 0)),
    pl.BlockSpec((D, D), lambda i: (0, 0)),
    pl.BlockSpec((D, D), lambda i: (0, 0)),
]


def kernel(nodes, first_node_idx, last_node_idx, candidate_indices,
           W_coord, b_coord, W1_w, W1_b, W2_w, W2_b):
    nodes_t = jnp.transpose(nodes, (1, 2, 0))
    cand_t = jnp.transpose(candidate_indices, (1, 0))
    wrows = jnp.concatenate(
        [W_coord, b_coord[None], W1_b[None], W2_b[None],
         jnp.zeros((3, D), jnp.float32)], axis=0)

    g0 = _sc_calls[0](nodes_t, first_node_idx, last_node_idx, cand_t)
    out_half0 = pl.pallas_call(
        _tc_dense0,
        grid=(NSTEP,),
        in_specs=_common_in_specs,
        out_specs=pl.BlockSpec((R, TB, D), lambda i: (0, i, 0)),
        out_shape=jax.ShapeDtypeStruct((R, B, D), jnp.float32),
    )(*g0, wrows, W1_w, W2_w)

    g1 = _sc_calls[1](nodes_t, first_node_idx, last_node_idx, cand_t)
    out_t = pl.pallas_call(
        _tc_dense1,
        grid=(NSTEP,),
        in_specs=[pl.BlockSpec(memory_space=pl.ANY)] + _common_in_specs,
        out_specs=pl.BlockSpec((R, TB, D), lambda i: (0, i + NSTEP, 0)),
        out_shape=jax.ShapeDtypeStruct((R, B, D), jnp.float32),
        input_output_aliases={0: 0},
    )(out_half0, *g1, wrows, W1_w, W2_w)

    return jnp.transpose(out_t, (1, 0, 2))
